# baseline (device time: 101230 ns/iter reference)
import jax
import jax.numpy as jnp
from jax import lax
from jax.experimental import pallas as pl
from jax.experimental.pallas import tpu as pltpu

N_DEV = 4
N_HOP = N_DEV - 1
N_SEG = 4
N_DIR = 2


def kernel(t, W):
    m_per, k = t.shape
    n = W.shape[1]
    m_c = m_per // N_DEV
    m_h = m_c // 2
    m_q = m_h // N_SEG

    n_sems_per_phase = N_DIR * N_HOP * N_SEG

    def body(t_ref, w_ref, out_ref, send_ref, recv_ref, ag_ref,
             send_sems, recv_sems):
        d = lax.axis_index("i")
        left = (d - 1) % N_DEV
        right = (d + 1) % N_DEV

        def row0(c, dirn, s):
            return c * m_c + dirn * m_h + s * m_q

        def t_seg(c, dirn, s):
            return t_ref[pl.ds(row0(c, dirn, s), m_q), :]

        def rs_c_send(dirn, h):
            return (d - 1 - h) % N_DEV if dirn == 0 else (d + 1 + h) % N_DEV

        def rs_c_recv(dirn, h):
            return (d - 2 - h) % N_DEV if dirn == 0 else (d + 2 + h) % N_DEV

        def ag_c_send(dirn, h):
            return (d - h) % N_DEV if dirn == 0 else (d + h) % N_DEV

        def ag_c_recv(dirn, h):
            return (d - 1 - h) % N_DEV if dirn == 0 else (d + 1 + h) % N_DEV

        def rs_desc(dirn, h, s):
            idx = dirn * (N_HOP * N_SEG) + h * N_SEG + s
            return pltpu.make_async_remote_copy(
                src_ref=send_ref.at[dirn, s],
                dst_ref=recv_ref.at[dirn, h, s],
                send_sem=send_sems.at[idx],
                recv_sem=recv_sems.at[idx],
                device_id=(right if dirn == 0 else left,),
                device_id_type=pl.DeviceIdType.MESH,
            )

        def ag_desc(dirn, h, s):
            idx = n_sems_per_phase + dirn * (N_HOP * N_SEG) + h * N_SEG + s
            rows = pl.ds(row0(ag_c_send(dirn, h), dirn, s), m_q)
            return pltpu.make_async_remote_copy(
                src_ref=ag_ref.at[rows, :],
                dst_ref=ag_ref.at[rows, :],
                send_sem=send_sems.at[idx],
                recv_sem=recv_sems.at[idx],
                device_id=(right if dirn == 0 else left,),
                device_id_type=pl.DeviceIdType.MESH,
            )

        barrier_sem = pltpu.get_barrier_semaphore()
        for nbr in (left, right):
            pl.semaphore_signal(
                barrier_sem, inc=1,
                device_id=(nbr,), device_id_type=pl.DeviceIdType.MESH,
            )
        for dirn in range(N_DIR):
            c = rs_c_send(dirn, 0)
            for s in range(N_SEG):
                send_ref[dirn, s] = t_seg(c, dirn, s).astype(jnp.bfloat16)
        pl.semaphore_wait(barrier_sem, 2)

        for dirn in range(N_DIR):
            for s in range(N_SEG):
                rs_desc(dirn, 0, s).start()

        w_bf = w_ref[:, :].astype(jnp.bfloat16)

        for h in range(1, N_HOP):
            for dirn in range(N_DIR):
                c = rs_c_send(dirn, h)
                for s in range(N_SEG):
                    prev = rs_desc(dirn, h - 1, s)
                    prev.wait_recv()
                    prev.wait_send()
                    send_ref[dirn, s] = (
                        recv_ref[dirn, h - 1, s]
                        + t_seg(c, dirn, s).astype(jnp.bfloat16)
                    )
                    rs_desc(dirn, h, s).start()

        for dirn in range(N_DIR):
            for s in range(N_SEG):
                rs_desc(dirn, N_HOP - 1, s).wait_recv()
                s_seg = (
                    recv_ref[dirn, N_HOP - 1, s]
                    + t_seg(d, dirn, s).astype(jnp.bfloat16)
                )
                y = lax.dot_general(
                    s_seg, w_bf, (((1,), (0,)), ((), ())),
                    preferred_element_type=jnp.float32,
                )
                rows = pl.ds(row0(d, dirn, s), m_q)
                ag_ref[rows, :] = y.astype(jnp.bfloat16)
                ag_desc(dirn, 0, s).start()

        for h in range(1, N_HOP):
            for dirn in range(N_DIR):
                for s in range(N_SEG):
                    ag_desc(dirn, h - 1, s).wait_recv()
                    ag_desc(dirn, h, s).start()
            for dirn in range(N_DIR):
                c = ag_c_recv(dirn, h - 1)
                for s in range(N_SEG):
                    rows = pl.ds(row0(c, dirn, s), m_q)
                    out_ref[rows, :] = ag_ref[rows, :].astype(jnp.float32)

        for dirn in range(N_DIR):
            c = ag_c_recv(dirn, N_HOP - 1)
            for s in range(N_SEG):
                ag_desc(dirn, N_HOP - 1, s).wait_recv()
                rows = pl.ds(row0(c, dirn, s), m_q)
                out_ref[rows, :] = ag_ref[rows, :].astype(jnp.float32)
            for s in range(N_SEG):
                rows = pl.ds(row0(d, dirn, s), m_q)
                out_ref[rows, :] = ag_ref[rows, :].astype(jnp.float32)

        for dirn in range(N_DIR):
            for s in range(N_SEG):
                rs_desc(dirn, N_HOP - 1, s).wait_send()
        for h in range(N_HOP):
            for dirn in range(N_DIR):
                for s in range(N_SEG):
                    ag_desc(dirn, h, s).wait_send()

    n_sems = 2 * n_sems_per_phase
    return pl.pallas_call(
        body,
        out_shape=jax.ShapeDtypeStruct((m_per, n), jnp.float32),
        in_specs=[
            pl.BlockSpec(memory_space=pltpu.VMEM),
            pl.BlockSpec(memory_space=pltpu.VMEM),
        ],
        out_specs=pl.BlockSpec(memory_space=pltpu.VMEM),
        scratch_shapes=[
            pltpu.VMEM((N_DIR, N_SEG, m_q, k), jnp.bfloat16),
            pltpu.VMEM((N_DIR, N_HOP, N_SEG, m_q, k), jnp.bfloat16),
            pltpu.VMEM((m_per, n), jnp.bfloat16),
            pltpu.SemaphoreType.DMA((n_sems,)),
            pltpu.SemaphoreType.DMA((n_sems,)),
        ],
        compiler_params=pltpu.CompilerParams(
            collective_id=0,
            vmem_limit_bytes=100 * 1024 * 1024,
        ),
    )(t, W)
